# baseline (speedup 1.0000x reference)
import math

import jax
import jax.numpy as jnp
import numpy as np
from jax import lax
from jax.experimental import pallas as pl
from jax.experimental.pallas import tpu as pltpu
from jax.experimental.pallas import tpu_sc as plsc

D_MODEL = 768
BATCH = 4
SEQ = 8192
N_TOK = BATCH * SEQ
NUM_WORKERS = 32
P_PER_W = SEQ // NUM_WORKERS
CP = 8
ROWS = BATCH * CP
N_CHUNKS = P_PER_W // CP
LANES = 16
SCALE = math.sqrt(float(D_MODEL))


def _pe_table() -> np.ndarray:
    pos = np.arange(SEQ, dtype=np.float32)[:, None]
    div = np.exp(
        np.arange(0, D_MODEL, 2, dtype=np.float32)
        * (-(math.log(10000.0) / D_MODEL))
    )
    angle = (pos * div).astype(np.float32)
    pe = np.empty((SEQ, D_MODEL), dtype=np.float32)
    pe[:, 0::2] = np.sin(angle)
    pe[:, 1::2] = np.cos(angle)
    return pe


_PE = _pe_table()


def _sc_embed(x_hbm, pe_hbm, lut_hbm, out_hbm,
              idx_v, rows0, rows1, pe0, pe1, out0, out1,
              gsem0, gsem1, psem0, psem1, wsem0, wsem1):
    rows = (rows0, rows1)
    pes = (pe0, pe1)
    outs = (out0, out1)
    gsems = (gsem0, gsem1)
    psems = (psem0, psem1)
    wsems = (wsem0, wsem1)

    wid = lax.axis_index("s") * 2 + lax.axis_index("c")
    p_base = wid * P_PER_W
    pltpu.sync_copy(x_hbm.at[wid], idx_v)

    def start_gather(c, b):
        pltpu.async_copy(lut_hbm.at[idx_v.at[c]], rows[b], gsems[b])
        pltpu.async_copy(
            pe_hbm.at[pl.ds(p_base + c * CP, CP)], pes[b], psems[b])

    def wait_gather(c, b):
        pltpu.make_async_copy(lut_hbm.at[idx_v.at[c]], rows[b], gsems[b]).wait()
        pltpu.make_async_copy(
            pe_hbm.at[pl.ds(p_base + c * CP, CP)], pes[b], psems[b]).wait()

    def out_copy(c, b, bb):
        return pltpu.make_async_copy(
            outs[b].at[pl.ds(bb * CP, CP)],
            out_hbm.at[pl.ds(bb * SEQ + p_base + c * CP, CP)],
            wsems[b])

    start_gather(0, 0)
    start_gather(1, 1)

    def step(c, b):
        wait_gather(c, b)

        @pl.when(c >= 2)
        def _():
            for bb in range(BATCH):
                out_copy(c - 2, b, bb).wait()

        UNROLL = 4

        def row_body(r, _):
            def col_body(k4, _):
                for u in range(UNROLL):
                    sl = pl.ds((k4 * UNROLL + u) * LANES, LANES)
                    pev = pes[b][r, sl]
                    for bb in range(BATCH):
                        outs[b][bb * CP + r, sl] = (
                            rows[b][bb * CP + r, sl] * SCALE + pev)
                return 0

            lax.fori_loop(0, D_MODEL // LANES // UNROLL, col_body, 0)
            return 0

        lax.fori_loop(0, CP, row_body, 0)
        for bb in range(BATCH):
            out_copy(c, b, bb).start()

        @pl.when(c + 2 < N_CHUNKS)
        def _():
            start_gather(c + 2, b)

    def pair(i, _):
        step(i * 2, 0)
        step(i * 2 + 1, 1)
        return 0

    lax.fori_loop(0, N_CHUNKS // 2, pair, 0)
    for bb in range(BATCH):
        out_copy(N_CHUNKS - 2, 0, bb).wait()
        out_copy(N_CHUNKS - 1, 1, bb).wait()


def kernel(x, lut):
    x_w = (x.astype(jnp.int32)
           .reshape(BATCH, NUM_WORKERS, N_CHUNKS, CP)
           .transpose(1, 2, 0, 3)
           .reshape(NUM_WORKERS, N_CHUNKS, ROWS))
    pe = jnp.asarray(_PE)
    run = pl.kernel(
        _sc_embed,
        out_type=jax.ShapeDtypeStruct((N_TOK, D_MODEL), jnp.float32),
        mesh=plsc.VectorSubcoreMesh(core_axis_name="c", subcore_axis_name="s"),
        scratch_types=[
            pltpu.VMEM((N_CHUNKS, ROWS), jnp.int32),
            pltpu.VMEM((ROWS, D_MODEL), jnp.float32),
            pltpu.VMEM((ROWS, D_MODEL), jnp.float32),
            pltpu.VMEM((CP, D_MODEL), jnp.float32),
            pltpu.VMEM((CP, D_MODEL), jnp.float32),
            pltpu.VMEM((ROWS, D_MODEL), jnp.float32),
            pltpu.VMEM((ROWS, D_MODEL), jnp.float32),
            pltpu.SemaphoreType.DMA,
            pltpu.SemaphoreType.DMA,
            pltpu.SemaphoreType.DMA,
            pltpu.SemaphoreType.DMA,
            pltpu.SemaphoreType.DMA,
            pltpu.SemaphoreType.DMA,
        ],
    )
    out = run(x_w, pe, lut)
    return out.reshape(BATCH, SEQ, D_MODEL)


# speedup vs baseline: 1.1776x; 1.1776x over previous
import math

import jax
import jax.numpy as jnp
import numpy as np
from jax import lax
from jax.experimental import pallas as pl
from jax.experimental.pallas import tpu as pltpu
from jax.experimental.pallas import tpu_sc as plsc

D_MODEL = 768
BATCH = 4
SEQ = 8192
N_TOK = BATCH * SEQ
NUM_WORKERS = 32
P_PER_W = SEQ // NUM_WORKERS
CP = 8
ROWS = BATCH * CP
N_CHUNKS = P_PER_W // CP
LANES = 16
SCALE = math.sqrt(float(D_MODEL))


def _pe_table() -> np.ndarray:
    pos = np.arange(SEQ, dtype=np.float32)[:, None]
    div = np.exp(
        np.arange(0, D_MODEL, 2, dtype=np.float32)
        * (-(math.log(10000.0) / D_MODEL))
    )
    angle = (pos * div).astype(np.float32)
    pe = np.empty((SEQ, D_MODEL), dtype=np.float32)
    pe[:, 0::2] = np.sin(angle)
    pe[:, 1::2] = np.cos(angle)
    return pe


_PE = _pe_table()


def _sc_embed(x_hbm, pe_hbm, lut_hbm, out_hbm,
              idx_v, rows0, rows1, pe0, pe1, out0, out1,
              gsem0, gsem1, psem0, psem1, wsem0, wsem1):
    rows = (rows0, rows1)
    pes = (pe0, pe1)
    outs = (out0, out1)
    gsems = (gsem0, gsem1)
    psems = (psem0, psem1)
    wsems = (wsem0, wsem1)

    wid = lax.axis_index("s") * 2 + lax.axis_index("c")
    p_base = wid * P_PER_W
    pltpu.sync_copy(x_hbm.at[wid], idx_v)

    def start_gather(c, b):
        pltpu.async_copy(lut_hbm.at[idx_v.at[c]], rows[b], gsems[b])
        pltpu.async_copy(
            pe_hbm.at[pl.ds(p_base + c * CP, CP)], pes[b], psems[b])

    def wait_gather(c, b):
        pltpu.make_async_copy(lut_hbm.at[idx_v.at[c]], rows[b], gsems[b]).wait()
        pltpu.make_async_copy(
            pe_hbm.at[pl.ds(p_base + c * CP, CP)], pes[b], psems[b]).wait()

    def out_copy(c, b, bb):
        return pltpu.make_async_copy(
            outs[b].at[pl.ds(bb * CP, CP)],
            out_hbm.at[pl.ds(bb * SEQ + p_base + c * CP, CP)],
            wsems[b])

    start_gather(0, 0)
    start_gather(1, 1)

    def step(c, b):
        wait_gather(c, b)

        @pl.when(c >= 2)
        def _():
            for bb in range(BATCH):
                out_copy(c - 2, b, bb).wait()

        def row_body(rr, _):
            r = lax.rem(rr, CP)
            for k in range(D_MODEL // LANES):
                sl = pl.ds(k * LANES, LANES)
                outs[b][rr, sl] = rows[b][rr, sl] * SCALE + pes[b][r, sl]
            return 0

        lax.fori_loop(0, ROWS, row_body, 0)
        for bb in range(BATCH):
            out_copy(c, b, bb).start()

        @pl.when(c + 2 < N_CHUNKS)
        def _():
            start_gather(c + 2, b)

    def pair(i, _):
        step(i * 2, 0)
        step(i * 2 + 1, 1)
        return 0

    lax.fori_loop(0, N_CHUNKS // 2, pair, 0)
    for bb in range(BATCH):
        out_copy(N_CHUNKS - 2, 0, bb).wait()
        out_copy(N_CHUNKS - 1, 1, bb).wait()


def kernel(x, lut):
    x_w = (x.astype(jnp.int32)
           .reshape(BATCH, NUM_WORKERS, N_CHUNKS, CP)
           .transpose(1, 2, 0, 3)
           .reshape(NUM_WORKERS, N_CHUNKS, ROWS))
    pe = jnp.asarray(_PE)
    run = pl.kernel(
        _sc_embed,
        out_type=jax.ShapeDtypeStruct((N_TOK, D_MODEL), jnp.float32),
        mesh=plsc.VectorSubcoreMesh(core_axis_name="c", subcore_axis_name="s"),
        scratch_types=[
            pltpu.VMEM((N_CHUNKS, ROWS), jnp.int32),
            pltpu.VMEM((ROWS, D_MODEL), jnp.float32),
            pltpu.VMEM((ROWS, D_MODEL), jnp.float32),
            pltpu.VMEM((CP, D_MODEL), jnp.float32),
            pltpu.VMEM((CP, D_MODEL), jnp.float32),
            pltpu.VMEM((ROWS, D_MODEL), jnp.float32),
            pltpu.VMEM((ROWS, D_MODEL), jnp.float32),
            pltpu.SemaphoreType.DMA,
            pltpu.SemaphoreType.DMA,
            pltpu.SemaphoreType.DMA,
            pltpu.SemaphoreType.DMA,
            pltpu.SemaphoreType.DMA,
            pltpu.SemaphoreType.DMA,
        ],
    )
    out = run(x_w, pe, lut)
    return out.reshape(BATCH, SEQ, D_MODEL)


# speedup vs baseline: 2.2056x; 1.8729x over previous
import math

import jax
import jax.numpy as jnp
import numpy as np
from jax import lax
from jax.experimental import pallas as pl
from jax.experimental.pallas import tpu as pltpu
from jax.experimental.pallas import tpu_sc as plsc

D_MODEL = 768
BATCH = 4
SEQ = 8192
N_TOK = BATCH * SEQ
NUM_WORKERS = 32
B_PER_W = N_TOK // NUM_WORKERS
CHUNK = 16
N_CHUNKS = B_PER_W // CHUNK
LANES = 16
SCALE = math.sqrt(float(D_MODEL))


def _pe_table() -> np.ndarray:
    pos = np.arange(SEQ, dtype=np.float32)[:, None]
    div = np.exp(
        np.arange(0, D_MODEL, 2, dtype=np.float32)
        * (-(math.log(10000.0) / D_MODEL))
    )
    angle = (pos * div).astype(np.float32)
    pe = np.empty((SEQ, D_MODEL), dtype=np.float32)
    pe[:, 0::2] = np.sin(angle)
    pe[:, 1::2] = np.cos(angle)
    return pe


_PE = _pe_table()


def _sc_embed(x_hbm, pe_hbm, lut_hbm, out_hbm,
              idx_v, rows0, rows1, pe0, pe1, out0, out1,
              gsem0, gsem1, psem0, psem1, wsem0, wsem1):
    rows = (rows0, rows1)
    pes = (pe0, pe1)
    outs = (out0, out1)
    gsems = (gsem0, gsem1)
    psems = (psem0, psem1)
    wsems = (wsem0, wsem1)

    wid = lax.axis_index("s") * 2 + lax.axis_index("c")
    base = wid * B_PER_W
    pltpu.sync_copy(x_hbm.at[wid], idx_v)
    pe_base = lax.rem(base, SEQ)

    def start_gather(c, b):
        pltpu.async_copy(lut_hbm.at[idx_v.at[c]], rows[b], gsems[b])
        pltpu.async_copy(
            pe_hbm.at[pl.ds(pe_base + c * CHUNK, CHUNK)], pes[b], psems[b])

    def wait_gather(c, b):
        pltpu.make_async_copy(lut_hbm.at[idx_v.at[c]], rows[b], gsems[b]).wait()
        pltpu.make_async_copy(
            pe_hbm.at[pl.ds(pe_base + c * CHUNK, CHUNK)], pes[b],
            psems[b]).wait()

    def out_copy(c, b):
        return pltpu.make_async_copy(
            outs[b], out_hbm.at[pl.ds(base + c * CHUNK, CHUNK)], wsems[b])

    start_gather(0, 0)
    start_gather(1, 1)

    def step(c, b):
        wait_gather(c, b)

        @pl.when(c >= 2)
        def _():
            out_copy(c - 2, b).wait()

        def row_body(r, _):
            for k in range(D_MODEL // LANES):
                sl = pl.ds(k * LANES, LANES)
                outs[b][r, sl] = rows[b][r, sl] * SCALE + pes[b][r, sl]
            return 0

        lax.fori_loop(0, CHUNK, row_body, 0)
        out_copy(c, b).start()

        @pl.when(c + 2 < N_CHUNKS)
        def _():
            start_gather(c + 2, b)

    def pair(i, _):
        step(i * 2, 0)
        step(i * 2 + 1, 1)
        return 0

    lax.fori_loop(0, N_CHUNKS // 2, pair, 0)
    out_copy(N_CHUNKS - 2, 0).wait()
    out_copy(N_CHUNKS - 1, 1).wait()


def kernel(x, lut):
    x_w = x.reshape(NUM_WORKERS, N_CHUNKS, CHUNK).astype(jnp.int32)
    pe = jnp.asarray(_PE)
    run = pl.kernel(
        _sc_embed,
        out_type=jax.ShapeDtypeStruct((N_TOK, D_MODEL), jnp.float32),
        mesh=plsc.VectorSubcoreMesh(core_axis_name="c", subcore_axis_name="s"),
        scratch_types=[
            pltpu.VMEM((N_CHUNKS, CHUNK), jnp.int32),
            pltpu.VMEM((CHUNK, D_MODEL), jnp.float32),
            pltpu.VMEM((CHUNK, D_MODEL), jnp.float32),
            pltpu.VMEM((CHUNK, D_MODEL), jnp.float32),
            pltpu.VMEM((CHUNK, D_MODEL), jnp.float32),
            pltpu.VMEM((CHUNK, D_MODEL), jnp.float32),
            pltpu.VMEM((CHUNK, D_MODEL), jnp.float32),
            pltpu.SemaphoreType.DMA,
            pltpu.SemaphoreType.DMA,
            pltpu.SemaphoreType.DMA,
            pltpu.SemaphoreType.DMA,
            pltpu.SemaphoreType.DMA,
            pltpu.SemaphoreType.DMA,
        ],
    )
    out = run(x_w, pe, lut)
    return out.reshape(BATCH, SEQ, D_MODEL)


# speedup vs baseline: 2.4247x; 1.0993x over previous
import math

import jax
import jax.numpy as jnp
import numpy as np
from jax import lax
from jax.experimental import pallas as pl
from jax.experimental.pallas import tpu as pltpu
from jax.experimental.pallas import tpu_sc as plsc

D_MODEL = 768
BATCH = 4
SEQ = 8192
N_TOK = BATCH * SEQ
NUM_WORKERS = 32
P_PER_W = SEQ // NUM_WORKERS
CP = 8
ROWS = BATCH * CP
N_CHUNKS = P_PER_W // CP
LANES = 16
SCALE = math.sqrt(float(D_MODEL))


def _pe_table() -> np.ndarray:
    pos = np.arange(SEQ, dtype=np.float32)[:, None]
    div = np.exp(
        np.arange(0, D_MODEL, 2, dtype=np.float32)
        * (-(math.log(10000.0) / D_MODEL))
    )
    angle = (pos * div).astype(np.float32)
    pe = np.empty((SEQ, D_MODEL), dtype=np.float32)
    pe[:, 0::2] = np.sin(angle)
    pe[:, 1::2] = np.cos(angle)
    return pe


_PE = _pe_table()


def _sc_embed(x_hbm, pe_hbm, lut_hbm, out_hbm,
              idx_v, rows0, rows1, pe0, pe1, out0, out1,
              gsem0, gsem1, psem0, psem1, wsem0, wsem1):
    rows = (rows0, rows1)
    pes = (pe0, pe1)
    outs = (out0, out1)
    gsems = (gsem0, gsem1)
    psems = (psem0, psem1)
    wsems = (wsem0, wsem1)

    wid = lax.axis_index("s") * 2 + lax.axis_index("c")
    p_base = wid * P_PER_W
    pltpu.sync_copy(x_hbm.at[wid], idx_v)

    def start_gather(c, b):
        pltpu.async_copy(lut_hbm.at[idx_v.at[c]], rows[b], gsems[b])
        pltpu.async_copy(
            pe_hbm.at[pl.ds(p_base + c * CP, CP)], pes[b], psems[b])

    def wait_gather(c, b):
        pltpu.make_async_copy(lut_hbm.at[idx_v.at[c]], rows[b], gsems[b]).wait()
        pltpu.make_async_copy(
            pe_hbm.at[pl.ds(p_base + c * CP, CP)], pes[b], psems[b]).wait()

    def out_copy(c, b, bb):
        return pltpu.make_async_copy(
            outs[b].at[pl.ds(bb * CP, CP)],
            out_hbm.at[pl.ds(bb * SEQ + p_base + c * CP, CP)],
            wsems[b])

    start_gather(0, 0)
    start_gather(1, 1)

    def step(c, b):
        wait_gather(c, b)

        @pl.when(c >= 2)
        def _():
            for bb in range(BATCH):
                out_copy(c - 2, b, bb).wait()

        for bb in range(BATCH):
            def row_body(r, _, _bb=bb):
                for k in range(D_MODEL // LANES):
                    sl = pl.ds(k * LANES, LANES)
                    outs[b][_bb * CP + r, sl] = (
                        rows[b][_bb * CP + r, sl] * SCALE + pes[b][r, sl])
                return 0

            lax.fori_loop(0, CP, row_body, 0)

        for bb in range(BATCH):
            out_copy(c, b, bb).start()

        @pl.when(c + 2 < N_CHUNKS)
        def _():
            start_gather(c + 2, b)

    def pair(i, _):
        step(i * 2, 0)
        step(i * 2 + 1, 1)
        return 0

    lax.fori_loop(0, N_CHUNKS // 2, pair, 0)
    for bb in range(BATCH):
        out_copy(N_CHUNKS - 2, 0, bb).wait()
        out_copy(N_CHUNKS - 1, 1, bb).wait()


def kernel(x, lut):
    x_w = (x.astype(jnp.int32)
           .reshape(BATCH, NUM_WORKERS, N_CHUNKS, CP)
           .transpose(1, 2, 0, 3)
           .reshape(NUM_WORKERS, N_CHUNKS, ROWS))
    pe = jnp.asarray(_PE)
    run = pl.kernel(
        _sc_embed,
        out_type=jax.ShapeDtypeStruct((N_TOK, D_MODEL), jnp.float32),
        mesh=plsc.VectorSubcoreMesh(core_axis_name="c", subcore_axis_name="s"),
        scratch_types=[
            pltpu.VMEM((N_CHUNKS, ROWS), jnp.int32),
            pltpu.VMEM((ROWS, D_MODEL), jnp.float32),
            pltpu.VMEM((ROWS, D_MODEL), jnp.float32),
            pltpu.VMEM((CP, D_MODEL), jnp.float32),
            pltpu.VMEM((CP, D_MODEL), jnp.float32),
            pltpu.VMEM((ROWS, D_MODEL), jnp.float32),
            pltpu.VMEM((ROWS, D_MODEL), jnp.float32),
            pltpu.SemaphoreType.DMA,
            pltpu.SemaphoreType.DMA,
            pltpu.SemaphoreType.DMA,
            pltpu.SemaphoreType.DMA,
            pltpu.SemaphoreType.DMA,
            pltpu.SemaphoreType.DMA,
        ],
    )
    out = run(x_w, pe, lut)
    return out.reshape(BATCH, SEQ, D_MODEL)


# speedup vs baseline: 2.7066x; 1.1163x over previous
import math

import jax
import jax.numpy as jnp
import numpy as np
from jax import lax
from jax.experimental import pallas as pl
from jax.experimental.pallas import tpu as pltpu
from jax.experimental.pallas import tpu_sc as plsc

D_MODEL = 768
BATCH = 4
SEQ = 8192
N_TOK = BATCH * SEQ
NUM_WORKERS = 32
P_PER_W = SEQ // NUM_WORKERS
CP = 8
ROWS = BATCH * CP
N_CHUNKS = P_PER_W // CP
LANES = 16
SCALE = math.sqrt(float(D_MODEL))


def _pe_table() -> np.ndarray:
    pos = np.arange(SEQ, dtype=np.float32)[:, None]
    div = np.exp(
        np.arange(0, D_MODEL, 2, dtype=np.float32)
        * (-(math.log(10000.0) / D_MODEL))
    )
    angle = (pos * div).astype(np.float32)
    pe = np.empty((SEQ, D_MODEL), dtype=np.float32)
    pe[:, 0::2] = np.sin(angle)
    pe[:, 1::2] = np.cos(angle)
    return pe


_PE = _pe_table()


def _sc_embed(x_hbm, pe_hbm, lut_hbm, out_hbm,
              idx_v, rows0, rows1, pe0, pe1, out0, out1,
              gsem0, gsem1, psem0, psem1, wsem0, wsem1):
    rows = (rows0, rows1)
    pes = (pe0, pe1)
    outs = (out0, out1)
    gsems = (gsem0, gsem1)
    psems = (psem0, psem1)
    wsems = (wsem0, wsem1)

    wid = lax.axis_index("s") * 2 + lax.axis_index("c")
    p_base = wid * P_PER_W
    pltpu.sync_copy(x_hbm.at[wid], idx_v)

    def start_gather(c, b):
        pltpu.async_copy(lut_hbm.at[idx_v.at[c]], rows[b], gsems[b])
        pltpu.async_copy(
            pe_hbm.at[pl.ds(p_base + c * CP, CP)], pes[b], psems[b])

    def wait_gather(c, b):
        pltpu.make_async_copy(lut_hbm.at[idx_v.at[c]], rows[b], gsems[b]).wait()
        pltpu.make_async_copy(
            pe_hbm.at[pl.ds(p_base + c * CP, CP)], pes[b], psems[b]).wait()

    def out_copy(c, b, bb):
        return pltpu.make_async_copy(
            outs[b].at[pl.ds(bb * CP, CP)],
            out_hbm.at[pl.ds(bb * SEQ + p_base + c * CP, CP)],
            wsems[b])

    start_gather(0, 0)
    start_gather(1, 1)

    def step(c, b):
        wait_gather(c, b)

        @pl.when(c >= 2)
        def _():
            for bb in range(BATCH):
                out_copy(c - 2, b, bb).wait()

        def col_body(k, _):
            sl = pl.ds(k * LANES, LANES)
            pev = [pes[b][r, sl] for r in range(CP)]
            for bb in range(BATCH):
                for r in range(CP):
                    rr = bb * CP + r
                    outs[b][rr, sl] = rows[b][rr, sl] * SCALE + pev[r]
            return 0

        lax.fori_loop(0, D_MODEL // LANES, col_body, 0)

        for bb in range(BATCH):
            out_copy(c, b, bb).start()

        @pl.when(c + 2 < N_CHUNKS)
        def _():
            start_gather(c + 2, b)

    def pair(i, _):
        step(i * 2, 0)
        step(i * 2 + 1, 1)
        return 0

    lax.fori_loop(0, N_CHUNKS // 2, pair, 0)
    for bb in range(BATCH):
        out_copy(N_CHUNKS - 2, 0, bb).wait()
        out_copy(N_CHUNKS - 1, 1, bb).wait()


def kernel(x, lut):
    x_w = (x.astype(jnp.int32)
           .reshape(BATCH, NUM_WORKERS, N_CHUNKS, CP)
           .transpose(1, 2, 0, 3)
           .reshape(NUM_WORKERS, N_CHUNKS, ROWS))
    pe = jnp.asarray(_PE)
    run = pl.kernel(
        _sc_embed,
        out_type=jax.ShapeDtypeStruct((N_TOK, D_MODEL), jnp.float32),
        mesh=plsc.VectorSubcoreMesh(core_axis_name="c", subcore_axis_name="s"),
        scratch_types=[
            pltpu.VMEM((N_CHUNKS, ROWS), jnp.int32),
            pltpu.VMEM((ROWS, D_MODEL), jnp.float32),
            pltpu.VMEM((ROWS, D_MODEL), jnp.float32),
            pltpu.VMEM((CP, D_MODEL), jnp.float32),
            pltpu.VMEM((CP, D_MODEL), jnp.float32),
            pltpu.VMEM((ROWS, D_MODEL), jnp.float32),
            pltpu.VMEM((ROWS, D_MODEL), jnp.float32),
            pltpu.SemaphoreType.DMA,
            pltpu.SemaphoreType.DMA,
            pltpu.SemaphoreType.DMA,
            pltpu.SemaphoreType.DMA,
            pltpu.SemaphoreType.DMA,
            pltpu.SemaphoreType.DMA,
        ],
    )
    out = run(x_w, pe, lut)
    return out.reshape(BATCH, SEQ, D_MODEL)


# speedup vs baseline: 2.7184x; 1.0043x over previous
import math

import jax
import jax.numpy as jnp
import numpy as np
from jax import lax
from jax.experimental import pallas as pl
from jax.experimental.pallas import tpu as pltpu
from jax.experimental.pallas import tpu_sc as plsc

D_MODEL = 768
BATCH = 4
SEQ = 8192
N_TOK = BATCH * SEQ
NUM_WORKERS = 32
P_PER_W = SEQ // NUM_WORKERS
CP = 8
ROWS = BATCH * CP
N_CHUNKS = P_PER_W // CP
LANES = 16
SCALE = math.sqrt(float(D_MODEL))


def _pe_table() -> np.ndarray:
    pos = np.arange(SEQ, dtype=np.float32)[:, None]
    div = np.exp(
        np.arange(0, D_MODEL, 2, dtype=np.float32)
        * (-(math.log(10000.0) / D_MODEL))
    )
    angle = (pos * div).astype(np.float32)
    pe = np.empty((SEQ, D_MODEL), dtype=np.float32)
    pe[:, 0::2] = np.sin(angle)
    pe[:, 1::2] = np.cos(angle)
    return pe


_PE = _pe_table()


def _sc_embed(x_hbm, pe_hbm, lut_hbm, out_hbm,
              idx_v, rows0, rows1, pe0, pe1, out0, out1,
              gsem0, gsem1, psem0, psem1, wsem0, wsem1):
    rows = (rows0, rows1)
    pes = (pe0, pe1)
    outs = (out0, out1)
    gsems = (gsem0, gsem1)
    psems = (psem0, psem1)
    wsems = (wsem0, wsem1)

    wid = lax.axis_index("s") * 2 + lax.axis_index("c")
    p_base = wid * P_PER_W
    pltpu.sync_copy(x_hbm.at[wid], idx_v)

    def start_gather(c, b):
        pltpu.async_copy(lut_hbm.at[idx_v.at[c]], rows[b], gsems[b])
        pltpu.async_copy(
            pe_hbm.at[pl.ds((p_base + c * CP) * D_MODEL, CP * D_MODEL)],
            pes[b], psems[b])

    def wait_gather(c, b):
        pltpu.make_async_copy(lut_hbm.at[idx_v.at[c]], rows[b], gsems[b]).wait()
        pltpu.make_async_copy(
            pe_hbm.at[pl.ds((p_base + c * CP) * D_MODEL, CP * D_MODEL)],
            pes[b], psems[b]).wait()

    def out_copy(c, b, bb):
        return pltpu.make_async_copy(
            outs[b].at[pl.ds(bb * CP, CP)],
            out_hbm.at[pl.ds(bb * SEQ + p_base + c * CP, CP)],
            wsems[b])

    start_gather(0, 0)
    start_gather(1, 1)

    def step(c, b):
        wait_gather(c, b)

        @pl.when(c >= 2)
        def _():
            for bb in range(BATCH):
                out_copy(c - 2, b, bb).wait()

        def col_body(k, _):
            sl = pl.ds(k * LANES, LANES)
            pev = [pes[b][pl.ds(r * D_MODEL + k * LANES, LANES)] for r in range(CP)]
            for bb in range(BATCH):
                for r in range(CP):
                    rr = bb * CP + r
                    outs[b][rr, sl] = rows[b][rr, sl] * SCALE + pev[r]
            return 0

        lax.fori_loop(0, D_MODEL // LANES, col_body, 0)

        for bb in range(BATCH):
            out_copy(c, b, bb).start()

        @pl.when(c + 2 < N_CHUNKS)
        def _():
            start_gather(c + 2, b)

    def pair(i, _):
        step(i * 2, 0)
        step(i * 2 + 1, 1)
        return 0

    lax.fori_loop(0, N_CHUNKS // 2, pair, 0)
    for bb in range(BATCH):
        out_copy(N_CHUNKS - 2, 0, bb).wait()
        out_copy(N_CHUNKS - 1, 1, bb).wait()


def kernel(x, lut):
    x_w = (x.astype(jnp.int32)
           .reshape(BATCH, NUM_WORKERS, N_CHUNKS, CP)
           .transpose(1, 2, 0, 3)
           .reshape(NUM_WORKERS, N_CHUNKS, ROWS))
    pe = jnp.asarray(_PE.reshape(-1))
    run = pl.kernel(
        _sc_embed,
        out_type=jax.ShapeDtypeStruct((N_TOK, D_MODEL), jnp.float32),
        mesh=plsc.VectorSubcoreMesh(core_axis_name="c", subcore_axis_name="s"),
        scratch_types=[
            pltpu.VMEM((N_CHUNKS, ROWS), jnp.int32),
            pltpu.VMEM((ROWS, D_MODEL), jnp.float32),
            pltpu.VMEM((ROWS, D_MODEL), jnp.float32),
            pltpu.VMEM((CP * D_MODEL,), jnp.float32),
            pltpu.VMEM((CP * D_MODEL,), jnp.float32),
            pltpu.VMEM((ROWS, D_MODEL), jnp.float32),
            pltpu.VMEM((ROWS, D_MODEL), jnp.float32),
            pltpu.SemaphoreType.DMA,
            pltpu.SemaphoreType.DMA,
            pltpu.SemaphoreType.DMA,
            pltpu.SemaphoreType.DMA,
            pltpu.SemaphoreType.DMA,
            pltpu.SemaphoreType.DMA,
        ],
    )
    out = run(x_w, pe, lut)
    return out.reshape(BATCH, SEQ, D_MODEL)
